# baseline (speedup 1.0000x reference)
import functools

import jax
import jax.numpy as jnp
from jax import lax
from jax.experimental import pallas as pl
from jax.experimental.pallas import tpu as pltpu
from jax.experimental.pallas import tpu_sc as plsc

_N = 10000
_E = 320000
_D = 64
_G = 100
_CLS = 15

_NC = 2
_NS = 16
_NW = _NC * _NS
_CH = 125
_EPW = _E // _NW
_NCH = _EPW // _CH
_NP = 10240
_RPT = _NP // _NS
_ZR = 128

@functools.cache
def _make_propagate(width):
    mesh = plsc.VectorSubcoreMesh(core_axis_name="c", subcore_axis_name="s",
                                  num_cores=_NC, num_subcores=_NS)

    @functools.partial(
        pl.kernel,
        out_type=jax.ShapeDtypeStruct((_NW, _RPT, width), jnp.float32),
        mesh=mesh,
        scratch_types=[
            pltpu.VMEM((_NCH, _CH), jnp.int32),
            pltpu.VMEM((_NCH, _CH), jnp.int32),
            [pltpu.VMEM((_CH, width), jnp.float32)] * 8,
            pltpu.VMEM((_ZR, width), jnp.float32),
            pltpu.VMEM_SHARED((_NP, width), jnp.float32),
            [pltpu.SemaphoreType.DMA] * 5,
            [pltpu.SemaphoreType.DMA] * 5,
        ],
        compiler_params=pltpu.CompilerParams(use_tc_tiling_on_sc=False),
    )
    def prop(s_hbm, src_hbm, dst_hbm, out_hbm, srcv, dstv, rows, zbuf, acc,
             gsem, ssem):
        c = lax.axis_index("c")
        s = lax.axis_index("s")
        wid = c * _NS + s
        pltpu.sync_copy(src_hbm.at[wid], srcv)
        pltpu.sync_copy(dst_hbm.at[wid], dstv)

        def _zrow(i, carry):
            def _zcol(j, carry2):
                zbuf[i, pl.ds(j * 16, 16)] = jnp.zeros((16,), jnp.float32)
                return carry2
            return lax.fori_loop(0, width // 16, _zcol, carry)
        lax.fori_loop(0, _ZR, _zrow, 0)

        def _zslice(k, carry):
            pltpu.sync_copy(zbuf, acc.at[pl.ds(s * _RPT + k * _ZR, _ZR)])
            return carry
        lax.fori_loop(0, _RPT // _ZR, _zslice, 0)
        plsc.subcore_barrier()

        nbuf = 5
        for b in range(nbuf):
            pltpu.async_copy(s_hbm.at[srcv.at[b]], rows[b], gsem[b])

        def _body(k, carry):
            for b in range(nbuf):
                c = nbuf * k + b
                pltpu.make_async_copy(s_hbm.at[srcv.at[c]], rows[b],
                                      gsem[b]).wait()
                pltpu.async_copy(rows[b], acc.at[dstv.at[c]], ssem[b],
                                 add=True)

                @pl.when(k < _NCH // nbuf - 1)
                def _():
                    pltpu.make_async_copy(rows[b], acc.at[dstv.at[c]],
                                          ssem[b]).wait()
                    pltpu.async_copy(s_hbm.at[srcv.at[c + nbuf]], rows[b],
                                     gsem[b])
            return carry
        lax.fori_loop(0, _NCH // nbuf, _body, 0)
        for b in range(nbuf):
            pltpu.make_async_copy(rows[b], acc.at[dstv.at[0]], ssem[b]).wait()
        plsc.subcore_barrier()

        pltpu.sync_copy(acc.at[pl.ds(s * _RPT, _RPT)], out_hbm.at[wid])

    return prop


def _propagate(s_tab, src2, dst2):
    return _make_propagate(s_tab.shape[1])(s_tab, src2, dst2)


_BR = 1000


def _matmul(x, W):
    din = x.shape[1]

    def body(x_ref, w_ref, t_ref):
        t_ref[...] = jnp.dot(x_ref[...], w_ref[...],
                             preferred_element_type=jnp.float32)

    return pl.pallas_call(
        body,
        grid=(_N // _BR,),
        in_specs=[
            pl.BlockSpec((_BR, din), lambda i: (i, 0)),
            pl.BlockSpec((din, _D), lambda i: (0, 0)),
        ],
        out_specs=pl.BlockSpec((_BR, _D), lambda i: (i, 0)),
        out_shape=jax.ShapeDtypeStruct((_N, _D), jnp.float32),
    )(x, W)


def _scale_dinv(t, degacc):
    def body(t_ref, d_ref, s_ref, dv_ref):
        dv = lax.rsqrt(d_ref[0, :, 0:1] + d_ref[1, :, 0:1] + 1.0)
        dv_ref[...] = dv
        s_ref[...] = t_ref[...] * dv

    return pl.pallas_call(
        body,
        grid=(_N // _BR,),
        in_specs=[
            pl.BlockSpec((_BR, _D), lambda i: (i, 0)),
            pl.BlockSpec((2, _BR, 16), lambda i: (0, i, 0)),
        ],
        out_specs=[
            pl.BlockSpec((_BR, _D), lambda i: (i, 0)),
            pl.BlockSpec((_BR, 1), lambda i: (i, 0)),
        ],
        out_shape=[
            jax.ShapeDtypeStruct((_N, _D), jnp.float32),
            jax.ShapeDtypeStruct((_N, 1), jnp.float32),
        ],
    )(t, degacc)


def _combine_matmul_scale(p, t_prev, b, dinv, W):
    def body(p_ref, t_ref, b_ref, dv_ref, w_ref, tn_ref, sn_ref):
        dv = dv_ref[...]
        a = jnp.maximum(
            dv * (p_ref[0] + p_ref[1]) + dv * dv * t_ref[...] + b_ref[...], 0.0)
        tn = jnp.dot(a, w_ref[...], preferred_element_type=jnp.float32)
        tn_ref[...] = tn
        sn_ref[...] = tn * dv

    return pl.pallas_call(
        body,
        grid=(_N // _BR,),
        in_specs=[
            pl.BlockSpec((2, _BR, _D), lambda i: (0, i, 0)),
            pl.BlockSpec((_BR, _D), lambda i: (i, 0)),
            pl.BlockSpec((1, _D), lambda i: (0, 0)),
            pl.BlockSpec((_BR, 1), lambda i: (i, 0)),
            pl.BlockSpec((_D, _D), lambda i: (0, 0)),
        ],
        out_specs=[pl.BlockSpec((_BR, _D), lambda i: (i, 0))] * 2,
        out_shape=[jax.ShapeDtypeStruct((_N, _D), jnp.float32)] * 2,
    )(p, t_prev, b, dinv, W)


def _pool_head(p, t_prev, b, dinv, batch2d, fc_W, fc_b):
    steps = _N // _BR

    def body(p_ref, t_ref, b_ref, dv_ref, bt_ref, fw_ref, fb_ref,
             acc_ref, out_ref):
        i = pl.program_id(0)
        dv = dv_ref[...]
        a = jnp.maximum(
            dv * (p_ref[0] + p_ref[1]) + dv * dv * t_ref[...] + b_ref[...], 0.0)
        aug = jnp.concatenate([a, jnp.ones((_BR, 1), jnp.float32)], axis=1)
        gids = lax.broadcasted_iota(jnp.int32, (_G, _BR), 0)
        oh = (gids == bt_ref[0]).astype(jnp.float32)
        part = jnp.dot(oh, aug, preferred_element_type=jnp.float32)

        @pl.when(i == 0)
        def _():
            acc_ref[...] = jnp.zeros_like(acc_ref)

        acc_ref[...] += part

        @pl.when(i == steps - 1)
        def _():
            accv = acc_ref[...]
            pooled = accv[:, :_D] / jnp.maximum(accv[:, _D:], 1.0)
            logits = jnp.dot(pooled, fw_ref[...],
                             preferred_element_type=jnp.float32) + fb_ref[...]
            m = jnp.max(logits, axis=1, keepdims=True)
            lse = jnp.log(jnp.sum(jnp.exp(logits - m), axis=1, keepdims=True)) + m
            out_ref[...] = logits - lse

    _, out = pl.pallas_call(
        body,
        grid=(steps,),
        in_specs=[
            pl.BlockSpec((2, _BR, _D), lambda i: (0, i, 0)),
            pl.BlockSpec((_BR, _D), lambda i: (i, 0)),
            pl.BlockSpec((1, _D), lambda i: (0, 0)),
            pl.BlockSpec((_BR, 1), lambda i: (i, 0)),
            pl.BlockSpec((1, 1, _BR), lambda i: (i, 0, 0)),
            pl.BlockSpec((_D, _CLS), lambda i: (0, 0)),
            pl.BlockSpec((1, _CLS), lambda i: (0, 0)),
        ],
        out_specs=[
            pl.BlockSpec((_G, _D + 1), lambda i: (0, 0)),
            pl.BlockSpec((_G, _CLS), lambda i: (0, 0)),
        ],
        out_shape=[
            jax.ShapeDtypeStruct((_G, _D + 1), jnp.float32),
            jax.ShapeDtypeStruct((_G, _CLS), jnp.float32),
        ],
    )(p, t_prev, b, dinv, batch2d, fc_W, fc_b)
    return out


def kernel(x, edge_index, batch, W0, b0, W1, b1, W2, b2, fc_W, fc_b):
    src2 = edge_index[0].reshape(_NW, _NCH, _CH)
    dst2 = edge_index[1].reshape(_NW, _NCH, _CH)
    ones_tab = jnp.ones((_N, 16), jnp.float32)

    t0 = _matmul(x, W0)
    deg = _propagate(ones_tab, src2, dst2).reshape(2, _NP, 16)
    s0, dinv = _scale_dinv(t0, deg)
    p0 = _propagate(s0, src2, dst2).reshape(2, _NP, _D)
    t1, s1 = _combine_matmul_scale(p0, t0, b0.reshape(1, _D), dinv, W1)
    p1 = _propagate(s1, src2, dst2).reshape(2, _NP, _D)
    t2, s2 = _combine_matmul_scale(p1, t1, b1.reshape(1, _D), dinv, W2)
    p2 = _propagate(s2, src2, dst2).reshape(2, _NP, _D)
    return _pool_head(p2, t2, b2.reshape(1, _D), dinv,
                      batch.reshape(_N // _BR, 1, _BR), fc_W,
                      fc_b.reshape(1, _CLS))


# speedup vs baseline: 1.0367x; 1.0367x over previous
import functools

import jax
import jax.numpy as jnp
from jax import lax
from jax.experimental import pallas as pl
from jax.experimental.pallas import tpu as pltpu
from jax.experimental.pallas import tpu_sc as plsc

_N = 10000
_E = 320000
_D = 64
_G = 100
_CLS = 15

_NC = 2
_NS = 16
_NW = _NC * _NS
_CH = 125
_EPW = _E // _NW
_NCH = _EPW // _CH
_NP = 10240
_RPT = _NP // _NS
_ZR = 128

@functools.cache
def _make_propagate(width):
    mesh = plsc.VectorSubcoreMesh(core_axis_name="c", subcore_axis_name="s",
                                  num_cores=_NC, num_subcores=_NS)

    @functools.partial(
        pl.kernel,
        out_type=jax.ShapeDtypeStruct((_NC, _NP, width), jnp.float32),
        mesh=mesh,
        scratch_types=[
            pltpu.VMEM((_NCH, _CH), jnp.int32),
            pltpu.VMEM((_NCH, _CH), jnp.int32),
            [pltpu.VMEM((_CH, width), jnp.float32)] * 8,
            pltpu.VMEM((_ZR, width), jnp.float32),
            pltpu.VMEM_SHARED((_NP, width), jnp.float32),
            [pltpu.SemaphoreType.DMA] * 5,
            [pltpu.SemaphoreType.DMA] * 5,
        ],
        compiler_params=pltpu.CompilerParams(use_tc_tiling_on_sc=False),
    )
    def prop(s_hbm, ei_hbm, out_hbm, srcv, dstv, rows, zbuf, acc,
             gsem, ssem):
        c = lax.axis_index("c")
        s = lax.axis_index("s")
        wid = c * _NS + s
        pltpu.sync_copy(ei_hbm.at[0, wid], srcv)
        pltpu.sync_copy(ei_hbm.at[1, wid], dstv)

        def _zrow(i, carry):
            def _zcol(j, carry2):
                zbuf[i, pl.ds(j * 16, 16)] = jnp.zeros((16,), jnp.float32)
                return carry2
            return lax.fori_loop(0, width // 16, _zcol, carry)
        lax.fori_loop(0, _ZR, _zrow, 0)

        def _zslice(k, carry):
            pltpu.sync_copy(zbuf, acc.at[pl.ds(s * _RPT + k * _ZR, _ZR)])
            return carry
        lax.fori_loop(0, _RPT // _ZR, _zslice, 0)
        plsc.subcore_barrier()

        nbuf = 5
        for b in range(nbuf):
            pltpu.async_copy(s_hbm.at[srcv.at[b]], rows[b], gsem[b])

        def _body(k, carry):
            for b in range(nbuf):
                c = nbuf * k + b
                pltpu.make_async_copy(s_hbm.at[srcv.at[c]], rows[b],
                                      gsem[b]).wait()
                pltpu.async_copy(rows[b], acc.at[dstv.at[c]], ssem[b],
                                 add=True)

                @pl.when(k < _NCH // nbuf - 1)
                def _():
                    pltpu.make_async_copy(rows[b], acc.at[dstv.at[c]],
                                          ssem[b]).wait()
                    pltpu.async_copy(s_hbm.at[srcv.at[c + nbuf]], rows[b],
                                     gsem[b])
            return carry
        lax.fori_loop(0, _NCH // nbuf, _body, 0)
        for b in range(nbuf):
            pltpu.make_async_copy(rows[b], acc.at[dstv.at[0]], ssem[b]).wait()
        plsc.subcore_barrier()

        pltpu.sync_copy(acc.at[pl.ds(s * _RPT, _RPT)],
                        out_hbm.at[c, pl.ds(s * _RPT, _RPT)])

    return prop


def _propagate(s_tab, ei4):
    return _make_propagate(s_tab.shape[1])(s_tab, ei4)


_BR = 1000


def _matmul(x, W):
    din = x.shape[1]

    def body(x_ref, w_ref, t_ref):
        t_ref[...] = jnp.dot(x_ref[...], w_ref[...],
                             preferred_element_type=jnp.float32)

    return pl.pallas_call(
        body,
        grid=(_N // _BR,),
        in_specs=[
            pl.BlockSpec((_BR, din), lambda i: (i, 0)),
            pl.BlockSpec((din, _D), lambda i: (0, 0)),
        ],
        out_specs=pl.BlockSpec((_BR, _D), lambda i: (i, 0)),
        out_shape=jax.ShapeDtypeStruct((_N, _D), jnp.float32),
    )(x, W)


def _scale_dinv(t, degacc):
    def body(t_ref, d_ref, s_ref, dv_ref):
        dv = lax.rsqrt(d_ref[0, :, 0:1] + d_ref[1, :, 0:1] + 1.0)
        dv_ref[...] = dv
        s_ref[...] = t_ref[...] * dv

    return pl.pallas_call(
        body,
        grid=(_N // _BR,),
        in_specs=[
            pl.BlockSpec((_BR, _D), lambda i: (i, 0)),
            pl.BlockSpec((2, _BR, 16), lambda i: (0, i, 0)),
        ],
        out_specs=[
            pl.BlockSpec((_BR, _D), lambda i: (i, 0)),
            pl.BlockSpec((_BR, 1), lambda i: (i, 0)),
        ],
        out_shape=[
            jax.ShapeDtypeStruct((_N, _D), jnp.float32),
            jax.ShapeDtypeStruct((_N, 1), jnp.float32),
        ],
    )(t, degacc)


def _combine_matmul_scale(p, t_prev, b, dinv, W):
    def body(p_ref, t_ref, b_ref, dv_ref, w_ref, tn_ref, sn_ref):
        dv = dv_ref[...]
        a = jnp.maximum(
            dv * (p_ref[0] + p_ref[1]) + dv * dv * t_ref[...] + b_ref[...], 0.0)
        tn = jnp.dot(a, w_ref[...], preferred_element_type=jnp.float32)
        tn_ref[...] = tn
        sn_ref[...] = tn * dv

    return pl.pallas_call(
        body,
        grid=(_N // _BR,),
        in_specs=[
            pl.BlockSpec((2, _BR, _D), lambda i: (0, i, 0)),
            pl.BlockSpec((_BR, _D), lambda i: (i, 0)),
            pl.BlockSpec((1, _D), lambda i: (0, 0)),
            pl.BlockSpec((_BR, 1), lambda i: (i, 0)),
            pl.BlockSpec((_D, _D), lambda i: (0, 0)),
        ],
        out_specs=[pl.BlockSpec((_BR, _D), lambda i: (i, 0))] * 2,
        out_shape=[jax.ShapeDtypeStruct((_N, _D), jnp.float32)] * 2,
    )(p, t_prev, b, dinv, W)


def _pool_head(p, t_prev, b, dinv, batch2d, fc_W, fc_b):
    steps = _N // _BR

    def body(p_ref, t_ref, b_ref, dv_ref, bt_ref, fw_ref, fb_ref,
             acc_ref, out_ref):
        i = pl.program_id(0)
        dv = dv_ref[...]
        a = jnp.maximum(
            dv * (p_ref[0] + p_ref[1]) + dv * dv * t_ref[...] + b_ref[...], 0.0)
        aug = jnp.concatenate([a, jnp.ones((_BR, 1), jnp.float32)], axis=1)
        gids = lax.broadcasted_iota(jnp.int32, (_G, _BR), 0)
        oh = (gids == bt_ref[0]).astype(jnp.float32)
        part = jnp.dot(oh, aug, preferred_element_type=jnp.float32)

        @pl.when(i == 0)
        def _():
            acc_ref[...] = jnp.zeros_like(acc_ref)

        acc_ref[...] += part

        @pl.when(i == steps - 1)
        def _():
            accv = acc_ref[...]
            pooled = accv[:, :_D] / jnp.maximum(accv[:, _D:], 1.0)
            logits = jnp.dot(pooled, fw_ref[...],
                             preferred_element_type=jnp.float32) + fb_ref[...]
            m = jnp.max(logits, axis=1, keepdims=True)
            lse = jnp.log(jnp.sum(jnp.exp(logits - m), axis=1, keepdims=True)) + m
            out_ref[...] = logits - lse

    _, out = pl.pallas_call(
        body,
        grid=(steps,),
        in_specs=[
            pl.BlockSpec((2, _BR, _D), lambda i: (0, i, 0)),
            pl.BlockSpec((_BR, _D), lambda i: (i, 0)),
            pl.BlockSpec((1, _D), lambda i: (0, 0)),
            pl.BlockSpec((_BR, 1), lambda i: (i, 0)),
            pl.BlockSpec((1, 1, _BR), lambda i: (i, 0, 0)),
            pl.BlockSpec((_D, _CLS), lambda i: (0, 0)),
            pl.BlockSpec((1, _CLS), lambda i: (0, 0)),
        ],
        out_specs=[
            pl.BlockSpec((_G, _D + 1), lambda i: (0, 0)),
            pl.BlockSpec((_G, _CLS), lambda i: (0, 0)),
        ],
        out_shape=[
            jax.ShapeDtypeStruct((_G, _D + 1), jnp.float32),
            jax.ShapeDtypeStruct((_G, _CLS), jnp.float32),
        ],
    )(p, t_prev, b, dinv, batch2d, fc_W, fc_b)
    return out


def kernel(x, edge_index, batch, W0, b0, W1, b1, W2, b2, fc_W, fc_b):
    ei4 = edge_index.reshape(2, _NW, _NCH, _CH)
    ones_tab = jnp.ones((_N, 16), jnp.float32)

    t0 = _matmul(x, W0)
    deg = _propagate(ones_tab, ei4)
    s0, dinv = _scale_dinv(t0, deg)
    p0 = _propagate(s0, ei4)
    t1, s1 = _combine_matmul_scale(p0, t0, b0.reshape(1, _D), dinv, W1)
    p1 = _propagate(s1, ei4)
    t2, s2 = _combine_matmul_scale(p1, t1, b1.reshape(1, _D), dinv, W2)
    p2 = _propagate(s2, ei4)
    return _pool_head(p2, t2, b2.reshape(1, _D), dinv,
                      batch.reshape(_N // _BR, 1, _BR), fc_W,
                      fc_b.reshape(1, _CLS))


# speedup vs baseline: 1.0524x; 1.0151x over previous
import functools

import jax
import jax.numpy as jnp
from jax import lax
from jax.experimental import pallas as pl
from jax.experimental.pallas import tpu as pltpu
from jax.experimental.pallas import tpu_sc as plsc

_N = 10000
_E = 320000
_D = 64
_G = 100
_CLS = 15

_NC = 2
_NS = 16
_NW = _NC * _NS
_CH = 125
_EPW = _E // _NW
_NCH = _EPW // _CH
_NP = 10240
_RPT = _NP // _NS
_ZR = 128

@functools.cache
def _make_propagate(width):
    mesh = plsc.VectorSubcoreMesh(core_axis_name="c", subcore_axis_name="s",
                                  num_cores=_NC, num_subcores=_NS)

    @functools.partial(
        pl.kernel,
        out_type=jax.ShapeDtypeStruct((_NC, _NP, width), jnp.float32),
        mesh=mesh,
        scratch_types=[
            pltpu.VMEM((_NCH, _CH), jnp.int32),
            pltpu.VMEM((_NCH, _CH), jnp.int32),
            [pltpu.VMEM((_CH, width), jnp.float32)] * 8,
            pltpu.VMEM((_ZR, width), jnp.float32),
            pltpu.VMEM_SHARED((_NP, width), jnp.float32),
            [pltpu.SemaphoreType.DMA] * 5,
            [pltpu.SemaphoreType.DMA] * 5,
        ],
        compiler_params=pltpu.CompilerParams(use_tc_tiling_on_sc=False),
    )
    def prop(s_hbm, ei_hbm, out_hbm, srcv, dstv, rows, zbuf, acc,
             gsem, ssem):
        c = lax.axis_index("c")
        s = lax.axis_index("s")
        wid = c * _NS + s
        pltpu.sync_copy(ei_hbm.at[0, wid], srcv)
        pltpu.sync_copy(ei_hbm.at[1, wid], dstv)

        def _zrow(i, carry):
            def _zcol(j, carry2):
                zbuf[i, pl.ds(j * 16, 16)] = jnp.zeros((16,), jnp.float32)
                return carry2
            return lax.fori_loop(0, width // 16, _zcol, carry)
        lax.fori_loop(0, _ZR, _zrow, 0)

        def _zslice(k, carry):
            pltpu.sync_copy(zbuf, acc.at[pl.ds(s * _RPT + k * _ZR, _ZR)])
            return carry
        lax.fori_loop(0, _RPT // _ZR, _zslice, 0)
        plsc.subcore_barrier()

        nbuf = 5
        for b in range(nbuf):
            pltpu.async_copy(s_hbm.at[srcv.at[b]], rows[b], gsem[b])

        def _body(k, carry):
            for b in range(nbuf):
                c = nbuf * k + b
                pltpu.make_async_copy(s_hbm.at[srcv.at[c]], rows[b],
                                      gsem[b]).wait()
                pltpu.async_copy(rows[b], acc.at[dstv.at[c]], ssem[b],
                                 add=True)

                @pl.when(k < _NCH // nbuf - 1)
                def _():
                    pltpu.make_async_copy(rows[b], acc.at[dstv.at[c]],
                                          ssem[b]).wait()
                    pltpu.async_copy(s_hbm.at[srcv.at[c + nbuf]], rows[b],
                                     gsem[b])
            return carry
        lax.fori_loop(0, _NCH // nbuf, _body, 0)
        for b in range(nbuf):
            pltpu.make_async_copy(rows[b], acc.at[dstv.at[0]], ssem[b]).wait()
        plsc.subcore_barrier()

        pltpu.sync_copy(acc.at[pl.ds(s * _RPT, _RPT)],
                        out_hbm.at[c, pl.ds(s * _RPT, _RPT)])

    return prop


def _propagate(s_tab, ei4):
    return _make_propagate(s_tab.shape[1])(s_tab, ei4)


_BR = 1000


def _matmul(x, W):
    din = x.shape[1]

    def body(x_ref, w_ref, t_ref):
        t_ref[...] = jnp.dot(x_ref[...], w_ref[...],
                             preferred_element_type=jnp.float32)

    return pl.pallas_call(
        body,
        grid=(_N // _BR,),
        in_specs=[
            pl.BlockSpec((_BR, din), lambda i: (i, 0)),
            pl.BlockSpec((din, _D), lambda i: (0, 0)),
        ],
        out_specs=pl.BlockSpec((_BR, _D), lambda i: (i, 0)),
        out_shape=jax.ShapeDtypeStruct((_N, _D), jnp.float32),
    )(x, W)


def _scale_dinv(t, degacc):
    def body(t_ref, d_ref, s_ref, dv_ref):
        dv = lax.rsqrt(d_ref[0, :, 0:1] + d_ref[1, :, 0:1] + 1.0)
        dv_ref[...] = dv
        s_ref[...] = t_ref[...] * dv

    return pl.pallas_call(
        body,
        grid=(_N // _BR,),
        in_specs=[
            pl.BlockSpec((_BR, _D), lambda i: (i, 0)),
            pl.BlockSpec((2, _BR, 16), lambda i: (0, i, 0)),
        ],
        out_specs=[
            pl.BlockSpec((_BR, _D), lambda i: (i, 0)),
            pl.BlockSpec((_BR, 1), lambda i: (i, 0)),
        ],
        out_shape=[
            jax.ShapeDtypeStruct((_N, _D), jnp.float32),
            jax.ShapeDtypeStruct((_N, 1), jnp.float32),
        ],
    )(t, degacc)


def _combine_matmul_scale(p, s_prev, b, dinv, W):
    def body(p_ref, s_ref, b_ref, dv_ref, w_ref, sn_ref):
        dv = dv_ref[...]
        a = jnp.maximum(
            dv * (p_ref[0] + p_ref[1] + s_ref[...]) + b_ref[...], 0.0)
        sn_ref[...] = jnp.dot(a, w_ref[...],
                              preferred_element_type=jnp.float32) * dv

    return pl.pallas_call(
        body,
        grid=(_N // _BR,),
        in_specs=[
            pl.BlockSpec((2, _BR, _D), lambda i: (0, i, 0)),
            pl.BlockSpec((_BR, _D), lambda i: (i, 0)),
            pl.BlockSpec((1, _D), lambda i: (0, 0)),
            pl.BlockSpec((_BR, 1), lambda i: (i, 0)),
            pl.BlockSpec((_D, _D), lambda i: (0, 0)),
        ],
        out_specs=pl.BlockSpec((_BR, _D), lambda i: (i, 0)),
        out_shape=jax.ShapeDtypeStruct((_N, _D), jnp.float32),
    )(p, s_prev, b, dinv, W)


def _pool_head(p, s_prev, b, dinv, batch2d, fc_W, fc_b):
    steps = _N // _BR

    def body(p_ref, s_ref, b_ref, dv_ref, bt_ref, fw_ref, fb_ref,
             acc_ref, out_ref):
        i = pl.program_id(0)
        dv = dv_ref[...]
        a = jnp.maximum(
            dv * (p_ref[0] + p_ref[1] + s_ref[...]) + b_ref[...], 0.0)
        aug = jnp.concatenate([a, jnp.ones((_BR, 1), jnp.float32)], axis=1)
        gids = lax.broadcasted_iota(jnp.int32, (_G, _BR), 0)
        oh = (gids == bt_ref[0]).astype(jnp.float32)
        part = jnp.dot(oh, aug, preferred_element_type=jnp.float32)

        @pl.when(i == 0)
        def _():
            acc_ref[...] = jnp.zeros_like(acc_ref)

        acc_ref[...] += part

        @pl.when(i == steps - 1)
        def _():
            accv = acc_ref[...]
            pooled = accv[:, :_D] / jnp.maximum(accv[:, _D:], 1.0)
            logits = jnp.dot(pooled, fw_ref[...],
                             preferred_element_type=jnp.float32) + fb_ref[...]
            m = jnp.max(logits, axis=1, keepdims=True)
            lse = jnp.log(jnp.sum(jnp.exp(logits - m), axis=1, keepdims=True)) + m
            out_ref[...] = logits - lse

    _, out = pl.pallas_call(
        body,
        grid=(steps,),
        in_specs=[
            pl.BlockSpec((2, _BR, _D), lambda i: (0, i, 0)),
            pl.BlockSpec((_BR, _D), lambda i: (i, 0)),
            pl.BlockSpec((1, _D), lambda i: (0, 0)),
            pl.BlockSpec((_BR, 1), lambda i: (i, 0)),
            pl.BlockSpec((1, 1, _BR), lambda i: (i, 0, 0)),
            pl.BlockSpec((_D, _CLS), lambda i: (0, 0)),
            pl.BlockSpec((1, _CLS), lambda i: (0, 0)),
        ],
        out_specs=[
            pl.BlockSpec((_G, _D + 1), lambda i: (0, 0)),
            pl.BlockSpec((_G, _CLS), lambda i: (0, 0)),
        ],
        out_shape=[
            jax.ShapeDtypeStruct((_G, _D + 1), jnp.float32),
            jax.ShapeDtypeStruct((_G, _CLS), jnp.float32),
        ],
    )(p, s_prev, b, dinv, batch2d, fc_W, fc_b)
    return out


def kernel(x, edge_index, batch, W0, b0, W1, b1, W2, b2, fc_W, fc_b):
    ei4 = edge_index.reshape(2, _NW, _NCH, _CH)
    ones_tab = jnp.ones((_N, 16), jnp.float32)

    t0 = _matmul(x, W0)
    deg = _propagate(ones_tab, ei4)
    s0, dinv = _scale_dinv(t0, deg)
    p0 = _propagate(s0, ei4)
    s1 = _combine_matmul_scale(p0, s0, b0.reshape(1, _D), dinv, W1)
    p1 = _propagate(s1, ei4)
    s2 = _combine_matmul_scale(p1, s1, b1.reshape(1, _D), dinv, W2)
    p2 = _propagate(s2, ei4)
    return _pool_head(p2, s2, b2.reshape(1, _D), dinv,
                      batch.reshape(_N // _BR, 1, _BR), fc_W,
                      fc_b.reshape(1, _CLS))


# speedup vs baseline: 1.1025x; 1.0476x over previous
import functools

import jax
import jax.numpy as jnp
from jax import lax
from jax.experimental import pallas as pl
from jax.experimental.pallas import tpu as pltpu
from jax.experimental.pallas import tpu_sc as plsc

_N = 10000
_E = 320000
_D = 64
_G = 100
_CLS = 15

_NC = 2
_NS = 16
_NW = _NC * _NS
_CH = 125
_EPW = _E // _NW
_NCH = _EPW // _CH
_NP = 10240
_RPT = _NP // _NS
_ZR = 128

@functools.cache
def _make_propagate(width):
    mesh = plsc.VectorSubcoreMesh(core_axis_name="c", subcore_axis_name="s",
                                  num_cores=_NC, num_subcores=_NS)

    @functools.partial(
        pl.kernel,
        out_type=jax.ShapeDtypeStruct((_NC, _NP, width), jnp.float32),
        mesh=mesh,
        scratch_types=[
            pltpu.VMEM((_NCH, _CH), jnp.int32),
            pltpu.VMEM((_NCH, _CH), jnp.int32),
            [pltpu.VMEM((_CH, width), jnp.float32)] * 8,
            pltpu.VMEM((_ZR, width), jnp.float32),
            pltpu.VMEM_SHARED((_NP, width), jnp.float32),
            [pltpu.SemaphoreType.DMA] * 5,
            [pltpu.SemaphoreType.DMA] * 5,
        ],
        compiler_params=pltpu.CompilerParams(use_tc_tiling_on_sc=False),
    )
    def prop(s_hbm, ei_hbm, out_hbm, srcv, dstv, rows, zbuf, acc,
             gsem, ssem):
        c = lax.axis_index("c")
        s = lax.axis_index("s")
        wid = c * _NS + s
        pltpu.sync_copy(ei_hbm.at[0, wid], srcv)
        pltpu.sync_copy(ei_hbm.at[1, wid], dstv)

        def _zrow(i, carry):
            def _zcol(j, carry2):
                zbuf[i, pl.ds(j * 16, 16)] = jnp.zeros((16,), jnp.float32)
                return carry2
            return lax.fori_loop(0, width // 16, _zcol, carry)
        lax.fori_loop(0, _ZR, _zrow, 0)

        def _zslice(k, carry):
            pltpu.sync_copy(zbuf, acc.at[pl.ds(s * _RPT + k * _ZR, _ZR)])
            return carry
        lax.fori_loop(0, _RPT // _ZR, _zslice, 0)
        plsc.subcore_barrier()

        nbuf = 5
        for b in range(nbuf):
            pltpu.async_copy(s_hbm.at[srcv.at[b]], rows[b], gsem[b])

        def _body(k, carry):
            for b in range(nbuf):
                c = nbuf * k + b
                pltpu.make_async_copy(s_hbm.at[srcv.at[c]], rows[b],
                                      gsem[b]).wait()
                pltpu.async_copy(rows[b], acc.at[dstv.at[c]], ssem[b],
                                 add=True)

                @pl.when(k < _NCH // nbuf - 1)
                def _():
                    pltpu.make_async_copy(rows[b], acc.at[dstv.at[c]],
                                          ssem[b]).wait()
                    pltpu.async_copy(s_hbm.at[srcv.at[c + nbuf]], rows[b],
                                     gsem[b])
            return carry
        lax.fori_loop(0, _NCH // nbuf, _body, 0)
        for b in range(nbuf):
            pltpu.make_async_copy(rows[b], acc.at[dstv.at[0]], ssem[b]).wait()
        plsc.subcore_barrier()

        pltpu.sync_copy(acc.at[pl.ds(s * _RPT, _RPT)],
                        out_hbm.at[c, pl.ds(s * _RPT, _RPT)])

    return prop


def _propagate(s_tab, ei4):
    return _make_propagate(s_tab.shape[1])(s_tab, ei4)


@functools.cache
def _make_deg():
    mesh = plsc.VectorSubcoreMesh(core_axis_name="c", subcore_axis_name="s",
                                  num_cores=_NC, num_subcores=_NS)

    @functools.partial(
        pl.kernel,
        out_type=jax.ShapeDtypeStruct((_NC, _NP, 16), jnp.float32),
        mesh=mesh,
        scratch_types=[
            pltpu.VMEM((_NCH, _CH), jnp.int32),
            pltpu.VMEM((_CH, 16), jnp.float32),
            pltpu.VMEM((_ZR, 16), jnp.float32),
            pltpu.VMEM_SHARED((_NP, 16), jnp.float32),
            [pltpu.SemaphoreType.DMA] * 5,
        ],
        compiler_params=pltpu.CompilerParams(use_tc_tiling_on_sc=False),
    )
    def degk(ei_hbm, out_hbm, dstv, obuf, zbuf, acc, ssem):
        c = lax.axis_index("c")
        s = lax.axis_index("s")
        pltpu.sync_copy(ei_hbm.at[1, c * _NS + s], dstv)

        def _fill(i, carry):
            obuf[i, pl.ds(0, 16)] = jnp.ones((16,), jnp.float32)
            return carry
        lax.fori_loop(0, _CH, _fill, 0)

        def _zfill(i, carry):
            zbuf[i, pl.ds(0, 16)] = jnp.zeros((16,), jnp.float32)
            return carry
        lax.fori_loop(0, _ZR, _zfill, 0)

        def _zslice(k, carry):
            pltpu.sync_copy(zbuf, acc.at[pl.ds(s * _RPT + k * _ZR, _ZR)])
            return carry
        lax.fori_loop(0, _RPT // _ZR, _zslice, 0)
        plsc.subcore_barrier()

        nbuf = 5

        def _body(k, carry):
            for b in range(nbuf):
                cc = nbuf * k + b

                @pl.when(k > 0)
                def _():
                    pltpu.make_async_copy(obuf, acc.at[dstv.at[0]],
                                          ssem[b]).wait()
                pltpu.async_copy(obuf, acc.at[dstv.at[cc]], ssem[b], add=True)
            return carry
        lax.fori_loop(0, _NCH // nbuf, _body, 0)
        for b in range(nbuf):
            pltpu.make_async_copy(obuf, acc.at[dstv.at[0]], ssem[b]).wait()
        plsc.subcore_barrier()

        pltpu.sync_copy(acc.at[pl.ds(s * _RPT, _RPT)],
                        out_hbm.at[c, pl.ds(s * _RPT, _RPT)])

    return degk


def _degrees(ei4):
    return _make_deg()(ei4)


_BR = 1000


def _matmul(x, W):
    din = x.shape[1]

    def body(x_ref, w_ref, t_ref):
        t_ref[...] = jnp.dot(x_ref[...], w_ref[...],
                             preferred_element_type=jnp.float32)

    return pl.pallas_call(
        body,
        grid=(_N // _BR,),
        in_specs=[
            pl.BlockSpec((_BR, din), lambda i: (i, 0)),
            pl.BlockSpec((din, _D), lambda i: (0, 0)),
        ],
        out_specs=pl.BlockSpec((_BR, _D), lambda i: (i, 0)),
        out_shape=jax.ShapeDtypeStruct((_N, _D), jnp.float32),
    )(x, W)


def _scale_dinv(t, degacc):
    def body(t_ref, d_ref, s_ref, dv_ref):
        dv = lax.rsqrt(d_ref[0, :, 0:1] + d_ref[1, :, 0:1] + 1.0)
        dv_ref[...] = dv
        s_ref[...] = t_ref[...] * dv

    return pl.pallas_call(
        body,
        grid=(_N // _BR,),
        in_specs=[
            pl.BlockSpec((_BR, _D), lambda i: (i, 0)),
            pl.BlockSpec((2, _BR, 16), lambda i: (0, i, 0)),
        ],
        out_specs=[
            pl.BlockSpec((_BR, _D), lambda i: (i, 0)),
            pl.BlockSpec((_BR, 1), lambda i: (i, 0)),
        ],
        out_shape=[
            jax.ShapeDtypeStruct((_N, _D), jnp.float32),
            jax.ShapeDtypeStruct((_N, 1), jnp.float32),
        ],
    )(t, degacc)


def _combine_matmul_scale(p, s_prev, b, dinv, W):
    def body(p_ref, s_ref, b_ref, dv_ref, w_ref, sn_ref):
        dv = dv_ref[...]
        a = jnp.maximum(
            dv * (p_ref[0] + p_ref[1] + s_ref[...]) + b_ref[...], 0.0)
        sn_ref[...] = jnp.dot(a, w_ref[...],
                              preferred_element_type=jnp.float32) * dv

    return pl.pallas_call(
        body,
        grid=(_N // _BR,),
        in_specs=[
            pl.BlockSpec((2, _BR, _D), lambda i: (0, i, 0)),
            pl.BlockSpec((_BR, _D), lambda i: (i, 0)),
            pl.BlockSpec((1, _D), lambda i: (0, 0)),
            pl.BlockSpec((_BR, 1), lambda i: (i, 0)),
            pl.BlockSpec((_D, _D), lambda i: (0, 0)),
        ],
        out_specs=pl.BlockSpec((_BR, _D), lambda i: (i, 0)),
        out_shape=jax.ShapeDtypeStruct((_N, _D), jnp.float32),
    )(p, s_prev, b, dinv, W)


def _pool_head(p, s_prev, b, dinv, batch2d, fc_W, fc_b):
    steps = _N // _BR

    def body(p_ref, s_ref, b_ref, dv_ref, bt_ref, fw_ref, fb_ref,
             acc_ref, out_ref):
        i = pl.program_id(0)
        dv = dv_ref[...]
        a = jnp.maximum(
            dv * (p_ref[0] + p_ref[1] + s_ref[...]) + b_ref[...], 0.0)
        aug = jnp.concatenate([a, jnp.ones((_BR, 1), jnp.float32)], axis=1)
        gids = lax.broadcasted_iota(jnp.int32, (_G, _BR), 0)
        oh = (gids == bt_ref[0]).astype(jnp.float32)
        part = jnp.dot(oh, aug, preferred_element_type=jnp.float32)

        @pl.when(i == 0)
        def _():
            acc_ref[...] = jnp.zeros_like(acc_ref)

        acc_ref[...] += part

        @pl.when(i == steps - 1)
        def _():
            accv = acc_ref[...]
            pooled = accv[:, :_D] / jnp.maximum(accv[:, _D:], 1.0)
            logits = jnp.dot(pooled, fw_ref[...],
                             preferred_element_type=jnp.float32) + fb_ref[...]
            m = jnp.max(logits, axis=1, keepdims=True)
            lse = jnp.log(jnp.sum(jnp.exp(logits - m), axis=1, keepdims=True)) + m
            out_ref[...] = logits - lse

    _, out = pl.pallas_call(
        body,
        grid=(steps,),
        in_specs=[
            pl.BlockSpec((2, _BR, _D), lambda i: (0, i, 0)),
            pl.BlockSpec((_BR, _D), lambda i: (i, 0)),
            pl.BlockSpec((1, _D), lambda i: (0, 0)),
            pl.BlockSpec((_BR, 1), lambda i: (i, 0)),
            pl.BlockSpec((1, 1, _BR), lambda i: (i, 0, 0)),
            pl.BlockSpec((_D, _CLS), lambda i: (0, 0)),
            pl.BlockSpec((1, _CLS), lambda i: (0, 0)),
        ],
        out_specs=[
            pl.BlockSpec((_G, _D + 1), lambda i: (0, 0)),
            pl.BlockSpec((_G, _CLS), lambda i: (0, 0)),
        ],
        out_shape=[
            jax.ShapeDtypeStruct((_G, _D + 1), jnp.float32),
            jax.ShapeDtypeStruct((_G, _CLS), jnp.float32),
        ],
    )(p, s_prev, b, dinv, batch2d, fc_W, fc_b)
    return out


def kernel(x, edge_index, batch, W0, b0, W1, b1, W2, b2, fc_W, fc_b):
    ei4 = edge_index.reshape(2, _NW, _NCH, _CH)

    t0 = _matmul(x, W0)
    deg = _degrees(ei4)
    s0, dinv = _scale_dinv(t0, deg)
    p0 = _propagate(s0, ei4)
    s1 = _combine_matmul_scale(p0, s0, b0.reshape(1, _D), dinv, W1)
    p1 = _propagate(s1, ei4)
    s2 = _combine_matmul_scale(p1, s1, b1.reshape(1, _D), dinv, W2)
    p2 = _propagate(s2, ei4)
    return _pool_head(p2, s2, b2.reshape(1, _D), dinv,
                      batch.reshape(_N // _BR, 1, _BR), fc_W,
                      fc_b.reshape(1, _CLS))
